# Initial kernel scaffold; baseline (speedup 1.0000x reference)
import jax
import jax.numpy as jnp
from jax.experimental import pallas as pl


def kernel(sentences, xg_table, whh, wout, bout):
    raise NotImplementedError("write your pallas kernel here")


# speedup vs baseline: 16.6248x; 16.6248x over previous
import functools

import jax
import jax.numpy as jnp
from jax import lax
from jax.experimental import pallas as pl
from jax.experimental.pallas import tpu as pltpu

_TAGSET = 45
_BB = 128
_UNROLL = 8


def _tagger_kernel(idx_ref, xg_tab_ref, whh_ref, wout_ref, bout_ref,
                   out_ref, xg_vmem, hs_vmem, sems, *, seq_len, hidden_dim,
                   block_b):
    T, H, BB = seq_len, hidden_dim, block_b
    nb = pl.program_id(0)
    rows = T * BB

    def issue(k, carry):
        base = k * _UNROLL
        t = base // BB
        for u in range(_UNROLL):
            j = base + u
            pltpu.make_async_copy(
                xg_tab_ref.at[pl.ds(idx_ref[nb, j], 1), :],
                xg_vmem.at[pl.ds(j, 1), :],
                sems.at[t]).start()
        return carry

    lax.fori_loop(0, rows // _UNROLL, issue, 0)

    whh = whh_ref[...]

    h = jnp.zeros((BB, H), jnp.float32)
    c = jnp.zeros((BB, H), jnp.float32)
    for t in range(T):
        pltpu.make_async_copy(
            xg_tab_ref.at[pl.ds(0, BB), :],
            xg_vmem.at[pl.ds(t * BB, BB), :],
            sems.at[t]).wait()
        xg_t = xg_vmem[pl.ds(t * BB, BB), :]
        if t == 0:
            gates = xg_t
        else:
            gates = xg_t + jnp.dot(h, whh,
                                   preferred_element_type=jnp.float32)
        sg = jax.nn.sigmoid(gates)
        i_g = sg[:, 0 * H:1 * H]
        f_g = sg[:, 1 * H:2 * H]
        g_g = 2.0 * sg[:, 2 * H:3 * H] - 1.0
        o_g = sg[:, 3 * H:4 * H]
        c = f_g * c + i_g * g_g
        h = o_g * jnp.tanh(c)
        hs_vmem[pl.ds(t * BB, BB), :] = h

    logits = (jnp.dot(hs_vmem[...], wout_ref[...],
                      preferred_element_type=jnp.float32) + bout_ref[...])
    m = jnp.max(logits, axis=1, keepdims=True)
    z = logits - m
    lse = jnp.log(jnp.sum(jnp.exp(z), axis=1, keepdims=True))
    out_ref[...] = z - lse


def kernel(sentences, xg_table, whh, wout, bout):
    B, T = sentences.shape
    H = whh.shape[0]
    VPAD = wout.shape[1]
    BB = _BB if B % _BB == 0 else B
    NB = B // BB

    idx = (sentences.astype(jnp.int32)
           .reshape(NB, BB, T).transpose(0, 2, 1).reshape(NB, T * BB))

    kern = functools.partial(_tagger_kernel, seq_len=T, hidden_dim=H,
                             block_b=BB)
    grid_spec = pltpu.PrefetchScalarGridSpec(
        num_scalar_prefetch=1,
        grid=(NB,),
        in_specs=[
            pl.BlockSpec(memory_space=pl.ANY),
            pl.BlockSpec((H, 4 * H), lambda nb, idx: (0, 0)),
            pl.BlockSpec((H, VPAD), lambda nb, idx: (0, 0)),
            pl.BlockSpec((1, VPAD), lambda nb, idx: (0, 0)),
        ],
        out_specs=pl.BlockSpec((None, T * BB, VPAD), lambda nb, idx: (nb, 0, 0)),
        scratch_shapes=[
            pltpu.VMEM((T * BB, 4 * H), jnp.float32),
            pltpu.VMEM((T * BB, H), jnp.float32),
            pltpu.SemaphoreType.DMA((T,)),
        ],
    )
    out = pl.pallas_call(
        kern,
        out_shape=jax.ShapeDtypeStruct((NB, T * BB, VPAD), jnp.float32),
        grid_spec=grid_spec,
        compiler_params=pltpu.CompilerParams(
            dimension_semantics=("parallel",),
            disable_bounds_checks=True),
    )(idx, xg_table, whh, wout, bout)

    out = (out.reshape(NB, T, BB, VPAD).transpose(0, 2, 1, 3)
           .reshape(B, T, VPAD))
    return out[:, :, :_TAGSET]
